# baseline (speedup 1.0000x reference)
import functools

import jax
import jax.numpy as jnp
from jax import lax
from jax.experimental import pallas as pl
from jax.experimental.pallas import tpu as pltpu
from jax.experimental.pallas import tpu_sc as plsc

N = 10000
E = 320000
D = 128
NUM_GRAPHS = 64

NC = 2
NS = 16
TILES = NC * NS

C = 128
NCH = 160
NCH0 = 160
NCH1 = NCH - NCH0
PER_TILE = NCH * C // 2
EPAD = NS * NCH * C
NPAD = 10016
ZSTRIPE = NPAD // NS
OSTRIPE = N // NS

def _sc_mesh():
    return plsc.VectorSubcoreMesh(
        core_axis_name="c", subcore_axis_name="s",
        num_cores=NC, num_subcores=NS)


def _deg_body(dst_hbm, out_hbm, idx_v, deg_v):
    c = lax.axis_index("c")
    s = lax.axis_index("s")
    wid = s * NC + c
    zeros16 = jnp.zeros((16,), jnp.float32)

    def zbody(i, carry):
        deg_v[pl.ds(i * 16, 16)] = zeros16
        return carry

    lax.fori_loop(0, NPAD // 16, zbody, 0)

    pltpu.sync_copy(dst_hbm.at[pl.ds(wid * PER_TILE, PER_TILE)], idx_v)
    ones16 = jnp.ones((16,), jnp.float32)

    def ebody(i, carry):
        idx = idx_v[pl.ds(i * 16, 16)]
        plsc.addupdate_scatter(deg_v, [idx], ones16)
        return carry

    lax.fori_loop(0, PER_TILE // 16, ebody, 0)
    pltpu.sync_copy(deg_v.at[pl.ds(0, N)], out_hbm.at[wid])


@functools.cache
def _deg_kernel():
    return pl.kernel(
        _deg_body,
        out_type=jax.ShapeDtypeStruct((TILES, N), jnp.float32),
        mesh=_sc_mesh(),
        scratch_types=[
            pltpu.VMEM((PER_TILE,), jnp.int32),
            pltpu.VMEM((NPAD,), jnp.float32),
        ],
        compiler_params=pltpu.CompilerParams(
            needs_layout_passes=False, use_tc_tiling_on_sc=False),
    )


def _agg_body(q_hbm, src_hbm, dst_hbm, zeros_hbm, out_hbm,
              src0, dst0, src1, dst1, rows0, rows1, agg_sh,
              gsem0, gsem1, ssem0, ssem1):
    c = lax.axis_index("c")
    s = lax.axis_index("s")

    base_e = jnp.where(c == 0, s * (NCH0 * C), NS * (NCH0 * C) + s * (NCH1 * C))
    nch = jnp.where(c == 0, NCH0, NCH1)

    @pl.when(nch > 0)
    def _():
        pltpu.sync_copy(zeros_hbm, agg_sh.at[pl.ds(s * ZSTRIPE, ZSTRIPE)])
    plsc.subcore_barrier()

    def _drain(rows, ssem):
        pltpu.make_async_copy(zeros_hbm.at[pl.ds(0, C)], rows, ssem).wait()

    def pbody(p, carry):
        off0 = base_e + (2 * p) * C

        @pl.when(p > 0)
        def _():
            _drain(rows0, ssem0)

        pltpu.sync_copy(src_hbm.at[pl.ds(off0, C)], src0)
        pltpu.sync_copy(dst_hbm.at[pl.ds(off0, C)], dst0)
        g0 = pltpu.async_copy(q_hbm.at[src0], rows0, gsem0)

        @pl.when(p > 0)
        def _():
            _drain(rows1, ssem1)

        pltpu.sync_copy(src_hbm.at[pl.ds(off0 + C, C)], src1)
        pltpu.sync_copy(dst_hbm.at[pl.ds(off0 + C, C)], dst1)
        g1 = pltpu.async_copy(q_hbm.at[src1], rows1, gsem1)

        g0.wait()
        pltpu.async_copy(rows0, agg_sh.at[dst0], ssem0, add=True)
        g1.wait()
        pltpu.async_copy(rows1, agg_sh.at[dst1], ssem1, add=True)
        return carry

    lax.fori_loop(0, nch // 2, pbody, 0)

    @pl.when(nch > 0)
    def _():
        _drain(rows0, ssem0)
        _drain(rows1, ssem1)

    plsc.subcore_barrier()

    @pl.when(nch > 0)
    def _():
        pltpu.sync_copy(agg_sh.at[pl.ds(s * OSTRIPE, OSTRIPE)],
                        out_hbm.at[pl.ds(s * OSTRIPE, OSTRIPE)])


@functools.cache
def _agg_kernel():
    return pl.kernel(
        _agg_body,
        out_type=jax.ShapeDtypeStruct((N, D), jnp.float32),
        mesh=_sc_mesh(),
        scratch_types=[
            pltpu.VMEM((C,), jnp.int32),
            pltpu.VMEM((C,), jnp.int32),
            pltpu.VMEM((C,), jnp.int32),
            pltpu.VMEM((C,), jnp.int32),
            pltpu.VMEM((C, D), jnp.float32),
            pltpu.VMEM((C, D), jnp.float32),
            pltpu.VMEM_SHARED((NPAD, D), jnp.float32),
            pltpu.SemaphoreType.DMA,
            pltpu.SemaphoreType.DMA,
            pltpu.SemaphoreType.DMA,
            pltpu.SemaphoreType.DMA,
        ],
        compiler_params=pltpu.CompilerParams(
            needs_layout_passes=False, use_tc_tiling_on_sc=False),
    )


def _dinv_body(parts_ref, dinv_ref):
    deg = 1.0 + jnp.sum(parts_ref[...], axis=0, keepdims=True)
    dinv_ref[...] = lax.rsqrt(deg)


_dinv_call = pl.pallas_call(
    _dinv_body, out_shape=jax.ShapeDtypeStruct((1, N), jnp.float32))


def _q1_body(x_ref, w_ref, dinv_ref, q_ref):
    q_ref[...] = dinv_ref[...] * jnp.dot(
        x_ref[...], w_ref[...], preferred_element_type=jnp.float32)


_q1_call = pl.pallas_call(
    _q1_body, out_shape=jax.ShapeDtypeStruct((N, D), jnp.float32))


def _bn_relu(p_ref, q_ref, dinv_ref, b_ref, g_ref, bt_ref):
    dinv = dinv_ref[...]
    pre = dinv * (p_ref[...] + q_ref[...]) + b_ref[...]
    mu = jnp.mean(pre, axis=0, keepdims=True)
    var = jnp.mean((pre - mu) * (pre - mu), axis=0, keepdims=True)
    h = g_ref[...] * (pre - mu) * lax.rsqrt(var + 1e-5) + bt_ref[...]
    return jnp.maximum(h, 0.0)


def _bnq_body(p_ref, q_ref, dinv_ref, b_ref, g_ref, bt_ref, w_ref, qn_ref):
    h = _bn_relu(p_ref, q_ref, dinv_ref, b_ref, g_ref, bt_ref)
    qn_ref[...] = dinv_ref[...] * jnp.dot(
        h, w_ref[...], preferred_element_type=jnp.float32)


_bnq_call = pl.pallas_call(
    _bnq_body, out_shape=jax.ShapeDtypeStruct((N, D), jnp.float32))


def _final_body(p_ref, q_ref, dinv_ref, b_ref, g_ref, bt_ref, batch_ref,
                m1_ref, mb1_ref, m2_ref, mb2_ref, m3_ref, mb3_ref,
                wo_ref, bo_ref, out_ref):
    h = _bn_relu(p_ref, q_ref, dinv_ref, b_ref, g_ref, bt_ref)
    gids = lax.broadcasted_iota(jnp.int32, (NUM_GRAPHS, 1), 0)
    oht = (gids == batch_ref[...]).astype(jnp.float32)
    pooled = jnp.dot(oht, h, preferred_element_type=jnp.float32)
    counts = jnp.dot(oht, jnp.ones((N, 1), jnp.float32),
                     preferred_element_type=jnp.float32)
    z = pooled / jnp.maximum(counts, 1.0)
    z = jnp.maximum(jnp.dot(z, m1_ref[...],
                            preferred_element_type=jnp.float32) + mb1_ref[...], 0.0)
    z = jnp.maximum(jnp.dot(z, m2_ref[...],
                            preferred_element_type=jnp.float32) + mb2_ref[...], 0.0)
    z = jnp.maximum(jnp.dot(z, m3_ref[...],
                            preferred_element_type=jnp.float32) + mb3_ref[...], 0.0)
    out_ref[...] = jnp.dot(z, wo_ref[...],
                           preferred_element_type=jnp.float32) + bo_ref[...]


_final_call = pl.pallas_call(
    _final_body, out_shape=jax.ShapeDtypeStruct((NUM_GRAPHS, 2), jnp.float32))


def kernel(x, edge_index, batch, W1, b1, g1, bt1, W2, b2, g2, bt2,
           W3, b3, g3, bt3, M1, mb1, M2, mb2, M3, mb3, Wo, bo):
    src = jnp.concatenate(
        [edge_index[0], jnp.zeros((EPAD - E,), jnp.int32)])
    dst = jnp.concatenate(
        [edge_index[1], jnp.full((EPAD - E,), N, jnp.int32)])
    zeros_block = jnp.zeros((ZSTRIPE, D), jnp.float32)

    deg_parts = _deg_kernel()(dst)
    dinv_row = _dinv_call(deg_parts)
    dinv_col = dinv_row.reshape(N, 1)

    agg = _agg_kernel()
    q = _q1_call(x, W1, dinv_col)
    p = agg(q, src, dst, zeros_block)
    q = _bnq_call(p, q, dinv_col, b1.reshape(1, D), g1.reshape(1, D),
                  bt1.reshape(1, D), W2)
    p = agg(q, src, dst, zeros_block)
    q = _bnq_call(p, q, dinv_col, b2.reshape(1, D), g2.reshape(1, D),
                  bt2.reshape(1, D), W3)
    p = agg(q, src, dst, zeros_block)

    return _final_call(
        p, q, dinv_col, b3.reshape(1, D), g3.reshape(1, D), bt3.reshape(1, D),
        batch.reshape(1, N), M1, mb1.reshape(1, D), M2, mb2.reshape(1, 64),
        M3, mb3.reshape(1, 32), Wo, bo.reshape(1, 2))


# speedup vs baseline: 1.3941x; 1.3941x over previous
import functools

import jax
import jax.numpy as jnp
from jax import lax
from jax.experimental import pallas as pl
from jax.experimental.pallas import tpu as pltpu
from jax.experimental.pallas import tpu_sc as plsc

N = 10000
E = 320000
D = 128
NUM_GRAPHS = 64

NC = 2
NS = 16
TILES = NC * NS

C = 128
NCH = 160
NCH0 = 128
NCH1 = NCH - NCH0
PER_TILE = NCH * C // 2
EPAD = NS * NCH * C
NPAD = 10016
ZSTRIPE = NPAD // NS
OSTRIPE = N // NS

def _sc_mesh():
    return plsc.VectorSubcoreMesh(
        core_axis_name="c", subcore_axis_name="s",
        num_cores=NC, num_subcores=NS)


def _deg_body(dst_hbm, out_hbm, idx_v, deg_v):
    c = lax.axis_index("c")
    s = lax.axis_index("s")
    wid = s * NC + c
    zeros16 = jnp.zeros((16,), jnp.float32)

    def zbody(i, carry):
        deg_v[pl.ds(i * 16, 16)] = zeros16
        return carry

    lax.fori_loop(0, NPAD // 16, zbody, 0)

    pltpu.sync_copy(dst_hbm.at[pl.ds(wid * PER_TILE, PER_TILE)], idx_v)
    ones16 = jnp.ones((16,), jnp.float32)

    def ebody(i, carry):
        idx = idx_v[pl.ds(i * 16, 16)]
        plsc.addupdate_scatter(deg_v, [idx], ones16)
        return carry

    lax.fori_loop(0, PER_TILE // 16, ebody, 0)
    pltpu.sync_copy(deg_v.at[pl.ds(0, N)], out_hbm.at[wid])


@functools.cache
def _deg_kernel():
    return pl.kernel(
        _deg_body,
        out_type=jax.ShapeDtypeStruct((TILES, N), jnp.float32),
        mesh=_sc_mesh(),
        scratch_types=[
            pltpu.VMEM((PER_TILE,), jnp.int32),
            pltpu.VMEM((NPAD,), jnp.float32),
        ],
        compiler_params=pltpu.CompilerParams(
            needs_layout_passes=False, use_tc_tiling_on_sc=False),
    )


def _agg_body(q_hbm, src_hbm, dst_hbm, zeros_hbm, out_hbm,
              src0, dst0, src1, dst1, rows0, rows1, agg_sh,
              gsem0, gsem1, ssem0, ssem1):
    c = lax.axis_index("c")
    s = lax.axis_index("s")

    base_e = jnp.where(c == 0, s * (NCH0 * C), NS * (NCH0 * C) + s * (NCH1 * C))
    nch = jnp.where(c == 0, NCH0, NCH1)

    @pl.when(nch > 0)
    def _():
        zeros16 = jnp.zeros((16,), jnp.float32)

        def zb(r, carry):
            for k in range(D // 16):
                rows0[r, pl.ds(k * 16, 16)] = zeros16
            return carry

        lax.fori_loop(0, C, zb, 0)
        for k in range(ZSTRIPE // C):
            pltpu.sync_copy(rows0, agg_sh.at[pl.ds(s * ZSTRIPE + k * C, C)])
        rem = ZSTRIPE - (ZSTRIPE // C) * C
        pltpu.sync_copy(
            rows0.at[pl.ds(0, rem)],
            agg_sh.at[pl.ds(s * ZSTRIPE + (ZSTRIPE // C) * C, rem)])
    plsc.subcore_barrier()

    def _drain(rows, ssem):
        pltpu.make_async_copy(zeros_hbm.at[pl.ds(0, C)], rows, ssem).wait()

    def pbody(p, carry):
        off0 = base_e + (2 * p) * C

        @pl.when(p > 0)
        def _():
            _drain(rows0, ssem0)

        pltpu.sync_copy(src_hbm.at[pl.ds(off0, C)], src0)
        pltpu.sync_copy(dst_hbm.at[pl.ds(off0, C)], dst0)
        g0 = pltpu.async_copy(q_hbm.at[src0], rows0, gsem0)

        @pl.when(p > 0)
        def _():
            _drain(rows1, ssem1)

        pltpu.sync_copy(src_hbm.at[pl.ds(off0 + C, C)], src1)
        pltpu.sync_copy(dst_hbm.at[pl.ds(off0 + C, C)], dst1)
        g1 = pltpu.async_copy(q_hbm.at[src1], rows1, gsem1)

        g0.wait()
        pltpu.async_copy(rows0, agg_sh.at[dst0], ssem0, add=True)
        g1.wait()
        pltpu.async_copy(rows1, agg_sh.at[dst1], ssem1, add=True)
        return carry

    lax.fori_loop(0, nch // 2, pbody, 0)

    @pl.when(nch > 0)
    def _():
        _drain(rows0, ssem0)
        _drain(rows1, ssem1)

    plsc.subcore_barrier()

    @pl.when(nch > 0)
    def _():
        pltpu.sync_copy(agg_sh.at[pl.ds(s * OSTRIPE, OSTRIPE)],
                        out_hbm.at[pl.ds(s * OSTRIPE, OSTRIPE)])


@functools.cache
def _agg_kernel():
    return pl.kernel(
        _agg_body,
        out_type=jax.ShapeDtypeStruct((N, D), jnp.float32),
        mesh=_sc_mesh(),
        scratch_types=[
            pltpu.VMEM((C,), jnp.int32),
            pltpu.VMEM((C,), jnp.int32),
            pltpu.VMEM((C,), jnp.int32),
            pltpu.VMEM((C,), jnp.int32),
            pltpu.VMEM((C, D), jnp.float32),
            pltpu.VMEM((C, D), jnp.float32),
            pltpu.VMEM_SHARED((NPAD, D), jnp.float32),
            pltpu.SemaphoreType.DMA,
            pltpu.SemaphoreType.DMA,
            pltpu.SemaphoreType.DMA,
            pltpu.SemaphoreType.DMA,
        ],
        compiler_params=pltpu.CompilerParams(
            needs_layout_passes=False, use_tc_tiling_on_sc=False),
    )


def _dinv_body(parts_ref, dinv_ref):
    deg = 1.0 + jnp.sum(parts_ref[...], axis=0, keepdims=True)
    dinv_ref[...] = lax.rsqrt(deg)


_dinv_call = pl.pallas_call(
    _dinv_body, out_shape=jax.ShapeDtypeStruct((1, N), jnp.float32))


def _q1_body(x_ref, w_ref, dinv_ref, q_ref):
    q_ref[...] = dinv_ref[...] * jnp.dot(
        x_ref[...], w_ref[...], preferred_element_type=jnp.float32)


_q1_call = pl.pallas_call(
    _q1_body, out_shape=jax.ShapeDtypeStruct((N, D), jnp.float32))


def _bn_relu(p_ref, q_ref, dinv_ref, b_ref, g_ref, bt_ref):
    dinv = dinv_ref[...]
    pre = dinv * (p_ref[...] + q_ref[...]) + b_ref[...]
    mu = jnp.mean(pre, axis=0, keepdims=True)
    var = jnp.mean((pre - mu) * (pre - mu), axis=0, keepdims=True)
    h = g_ref[...] * (pre - mu) * lax.rsqrt(var + 1e-5) + bt_ref[...]
    return jnp.maximum(h, 0.0)


def _bnq_body(p_ref, q_ref, dinv_ref, b_ref, g_ref, bt_ref, w_ref, qn_ref):
    h = _bn_relu(p_ref, q_ref, dinv_ref, b_ref, g_ref, bt_ref)
    qn_ref[...] = dinv_ref[...] * jnp.dot(
        h, w_ref[...], preferred_element_type=jnp.float32)


_bnq_call = pl.pallas_call(
    _bnq_body, out_shape=jax.ShapeDtypeStruct((N, D), jnp.float32))


def _final_body(p_ref, q_ref, dinv_ref, b_ref, g_ref, bt_ref, batch_ref,
                m1_ref, mb1_ref, m2_ref, mb2_ref, m3_ref, mb3_ref,
                wo_ref, bo_ref, out_ref):
    h = _bn_relu(p_ref, q_ref, dinv_ref, b_ref, g_ref, bt_ref)
    gids = lax.broadcasted_iota(jnp.int32, (NUM_GRAPHS, 1), 0)
    oht = (gids == batch_ref[...]).astype(jnp.float32)
    pooled = jnp.dot(oht, h, preferred_element_type=jnp.float32)
    counts = jnp.dot(oht, jnp.ones((N, 1), jnp.float32),
                     preferred_element_type=jnp.float32)
    z = pooled / jnp.maximum(counts, 1.0)
    z = jnp.maximum(jnp.dot(z, m1_ref[...],
                            preferred_element_type=jnp.float32) + mb1_ref[...], 0.0)
    z = jnp.maximum(jnp.dot(z, m2_ref[...],
                            preferred_element_type=jnp.float32) + mb2_ref[...], 0.0)
    z = jnp.maximum(jnp.dot(z, m3_ref[...],
                            preferred_element_type=jnp.float32) + mb3_ref[...], 0.0)
    out_ref[...] = jnp.dot(z, wo_ref[...],
                           preferred_element_type=jnp.float32) + bo_ref[...]


_final_call = pl.pallas_call(
    _final_body, out_shape=jax.ShapeDtypeStruct((NUM_GRAPHS, 2), jnp.float32))


def kernel(x, edge_index, batch, W1, b1, g1, bt1, W2, b2, g2, bt2,
           W3, b3, g3, bt3, M1, mb1, M2, mb2, M3, mb3, Wo, bo):
    src = jnp.concatenate(
        [edge_index[0], jnp.zeros((EPAD - E,), jnp.int32)])
    dst = jnp.concatenate(
        [edge_index[1], jnp.full((EPAD - E,), N, jnp.int32)])
    zeros_block = jnp.zeros((ZSTRIPE, D), jnp.float32)

    deg_parts = _deg_kernel()(dst)
    dinv_row = _dinv_call(deg_parts)
    dinv_col = dinv_row.reshape(N, 1)

    agg = _agg_kernel()
    q = _q1_call(x, W1, dinv_col)
    p = agg(q, src, dst, zeros_block)
    q = _bnq_call(p, q, dinv_col, b1.reshape(1, D), g1.reshape(1, D),
                  bt1.reshape(1, D), W2)
    p = agg(q, src, dst, zeros_block)
    q = _bnq_call(p, q, dinv_col, b2.reshape(1, D), g2.reshape(1, D),
                  bt2.reshape(1, D), W3)
    p = agg(q, src, dst, zeros_block)

    return _final_call(
        p, q, dinv_col, b3.reshape(1, D), g3.reshape(1, D), bt3.reshape(1, D),
        batch.reshape(1, N), M1, mb1.reshape(1, D), M2, mb2.reshape(1, 64),
        M3, mb3.reshape(1, 32), Wo, bo.reshape(1, 2))
